# baseline (speedup 1.0000x reference)
import functools
import math

import jax
import jax.numpy as jnp
from jax import lax
from jax.experimental import pallas as pl
from jax.experimental.pallas import tpu as pltpu
from jax.experimental.pallas import tpu_sc as plsc

NUM_BUCKETS = 32
MAX_DISTANCE = 128
N_HEADS = 16

Q_LEN = 2048
K_LEN = 2048
N_RHO = 16
N_K0 = 16
SLAB_W = 128 * (N_K0 - 1) + K_LEN
C_SRC = 4224


def _table_body(wt_ref, out_ref, vtab_ref):
    c = lax.broadcasted_iota(jnp.int32, (1, C_SRC), 1)
    d = c - (Q_LEN - 1)
    nb = NUM_BUCKETS // 2
    max_exact = nb // 2
    bucket = jnp.where(d > 0, nb, 0)
    r = jnp.abs(d)
    is_small = r < max_exact
    rp_safe = jnp.maximum(r, 1).astype(jnp.float32)
    large = max_exact + (
        jnp.log(rp_safe / max_exact)
        / math.log(MAX_DISTANCE / max_exact)
        * (nb - max_exact)
    ).astype(jnp.int32)
    large = jnp.minimum(large, nb - 1)
    bucket = bucket + jnp.where(is_small, r, large)

    bkt = jnp.broadcast_to(bucket, (N_HEADS, C_SRC))
    vtab = jnp.zeros((N_HEADS, C_SRC), jnp.float32)
    for b in range(NUM_BUCKETS):
        vtab = jnp.where(bkt == b, wt_ref[:, b : b + 1], vtab)
    vtab_ref[...] = vtab
    for p in range(N_RHO):
        rho = 8 * p + 7
        for rr in range(8):
            out_ref[:, p, rr, :] = vtab_ref[:, rho - rr : rho - rr + SLAB_W]


def _build_table(W):
    wt = W.T
    return pl.pallas_call(
        _table_body,
        out_shape=jax.ShapeDtypeStruct((N_HEADS, N_RHO, 8, SLAB_W), jnp.float32),
        scratch_shapes=[pltpu.VMEM((N_HEADS, C_SRC), jnp.float32)],
    )(wt)


@functools.lru_cache(maxsize=1)
def _expander():
    mesh = plsc.VectorSubcoreMesh(core_axis_name="c", subcore_axis_name="s")
    qkv_sds = jax.ShapeDtypeStruct((2, Q_LEN, 1024), jnp.float32)

    @functools.partial(
        pl.kernel,
        mesh=mesh,
        out_type=(
            jax.ShapeDtypeStruct((1, N_HEADS, Q_LEN, K_LEN), jnp.float32),
            qkv_sds,
            qkv_sds,
            qkv_sds,
        ),
        scratch_types=[
            pltpu.VMEM((8, SLAB_W), jnp.float32),
            pltpu.SemaphoreType.DMA,
            pltpu.SemaphoreType.DMA,
        ],
    )
    def expand(tdb_hbm, q_hbm, k_hbm, v_hbm,
               out_hbm, qo_hbm, ko_hbm, vo_hbm, slab_ref, sem, qkv_sem):
        wid = lax.axis_index("s") * 2 + lax.axis_index("c")
        r0 = pl.multiple_of(64 * wid, 64)
        qkv_copies = []
        for src, dst in ((q_hbm, qo_hbm), (k_hbm, ko_hbm), (v_hbm, vo_hbm)):
            for b in range(2):
                qkv_copies.append(
                    pltpu.async_copy(
                        src.at[b, pl.ds(r0, 64), :],
                        dst.at[b, pl.ds(r0, 64), :],
                        qkv_sem,
                    )
                )
        for n in range(8):
            t = wid * 8 + n
            h = t // N_RHO
            pidx = t % N_RHO
            pltpu.sync_copy(tdb_hbm.at[h, pidx], slab_ref)
            copies = []
            for k0 in range(N_K0):
                i0 = pl.multiple_of(8 * (255 - pidx - 16 * k0), 8)
                copies.append(
                    pltpu.async_copy(
                        slab_ref.at[:, pl.ds(128 * k0, K_LEN)],
                        out_hbm.at[0, h, pl.ds(i0, 8), :],
                        sem,
                    )
                )
            for cp in copies:
                cp.wait()
        for cp in qkv_copies:
            cp.wait()

    return expand


def kernel(q, k, v, W):
    tdb = _build_table(W)
    bias, qo, ko, vo = _expander()(tdb, q, k, v)
    return (qo, ko, vo, bias)


# speedup vs baseline: 8.4568x; 8.4568x over previous
import functools
import math

import jax
import jax.numpy as jnp
from jax import lax
from jax.experimental import pallas as pl
from jax.experimental.pallas import tpu as pltpu
from jax.experimental.pallas import tpu_sc as plsc

NUM_BUCKETS = 32
MAX_DISTANCE = 128
N_HEADS = 16

Q_LEN = 2048
K_LEN = 2048
N_RHO = 16
N_K0 = 16
SLAB_W = 128 * (N_K0 - 1) + K_LEN
C_SRC = 4224
CHUNK_R = 32
NCHUNK_PER_ARR = 4


def _table_body(wt_ref, out_ref, vtab_ref):
    c = lax.broadcasted_iota(jnp.int32, (1, C_SRC), 1)
    d = c - (Q_LEN - 1)
    nb = NUM_BUCKETS // 2
    max_exact = nb // 2
    bucket = jnp.where(d > 0, nb, 0)
    r = jnp.abs(d)
    is_small = r < max_exact
    rp_safe = jnp.maximum(r, 1).astype(jnp.float32)
    large = max_exact + (
        jnp.log(rp_safe / max_exact)
        / math.log(MAX_DISTANCE / max_exact)
        * (nb - max_exact)
    ).astype(jnp.int32)
    large = jnp.minimum(large, nb - 1)
    bucket = bucket + jnp.where(is_small, r, large)

    bkt = jnp.broadcast_to(bucket, (N_HEADS, C_SRC))
    vtab = jnp.zeros((N_HEADS, C_SRC), jnp.float32)
    for b in range(NUM_BUCKETS):
        vtab = jnp.where(bkt == b, wt_ref[:, b : b + 1], vtab)
    vtab_ref[...] = vtab
    for p in range(N_RHO):
        rho = 8 * p + 7
        for rr in range(8):
            out_ref[:, p, rr, :] = vtab_ref[:, rho - rr : rho - rr + SLAB_W]


def _build_table(W):
    wt = W.T
    return pl.pallas_call(
        _table_body,
        out_shape=jax.ShapeDtypeStruct((N_HEADS, N_RHO, 8, SLAB_W), jnp.float32),
        scratch_shapes=[pltpu.VMEM((N_HEADS, C_SRC), jnp.float32)],
    )(wt)


@functools.lru_cache(maxsize=1)
def _expander():
    mesh = plsc.VectorSubcoreMesh(core_axis_name="c", subcore_axis_name="s")
    qkv_sds = jax.ShapeDtypeStruct((2, Q_LEN, 1024), jnp.float32)

    @functools.partial(
        pl.kernel,
        mesh=mesh,
        out_type=(
            jax.ShapeDtypeStruct((1, N_HEADS, Q_LEN, K_LEN), jnp.float32),
            qkv_sds,
            qkv_sds,
            qkv_sds,
        ),
        scratch_types=[
            pltpu.VMEM((8, SLAB_W), jnp.float32),
            pltpu.VMEM((CHUNK_R, 1024), jnp.float32),
            pltpu.VMEM((CHUNK_R, 1024), jnp.float32),
            pltpu.SemaphoreType.DMA,
            pltpu.SemaphoreType.DMA,
            pltpu.SemaphoreType.DMA,
        ],
    )
    def expand(tdb_hbm, q_hbm, k_hbm, v_hbm,
               out_hbm, qo_hbm, ko_hbm, vo_hbm,
               slab_ref, cbuf0, cbuf1, sem, in_sem, out_sem):
        wid = lax.axis_index("s") * 2 + lax.axis_index("c")
        cbufs = (cbuf0, cbuf1)
        chunks = []
        for src, dst in ((q_hbm, qo_hbm), (k_hbm, ko_hbm), (v_hbm, vo_hbm)):
            for mm in range(NCHUNK_PER_ARR):
                br = wid * NCHUNK_PER_ARR + mm
                b = br // (Q_LEN // CHUNK_R)
                row0 = pl.multiple_of(CHUNK_R * (br % (Q_LEN // CHUNK_R)), CHUNK_R)
                chunks.append((src, dst, b, row0))

        def start_in(ci):
            src, _, b, row0 = chunks[ci]
            return pltpu.async_copy(
                src.at[b, pl.ds(row0, CHUNK_R), :], cbufs[ci % 2], in_sem)

        def start_out(ci):
            _, dst, b, row0 = chunks[ci]
            return pltpu.async_copy(
                cbufs[ci % 2], dst.at[b, pl.ds(row0, CHUNK_R), :], out_sem)

        ins = {0: start_in(0), 1: start_in(1)}

        for n in range(8):
            t = wid * 8 + n
            h = t // N_RHO
            pidx = t % N_RHO
            pltpu.sync_copy(tdb_hbm.at[h, pidx], slab_ref)
            copies = []
            for k0 in range(N_K0):
                i0 = pl.multiple_of(8 * (255 - pidx - 16 * k0), 8)
                copies.append(
                    pltpu.async_copy(
                        slab_ref.at[:, pl.ds(128 * k0, K_LEN)],
                        out_hbm.at[0, h, pl.ds(i0, 8), :],
                        sem,
                    )
                )
            for cp in copies:
                cp.wait()

        outs = {}
        for ci in range(len(chunks)):
            ins[ci].wait()
            outs[ci] = start_out(ci)
            if ci + 2 < len(chunks):
                outs[ci].wait()
                ins[ci + 2] = start_in(ci + 2)
        for ci in (len(chunks) - 2, len(chunks) - 1):
            outs[ci].wait()

    return expand


def kernel(q, k, v, W):
    tdb = _build_table(W)
    bias, qo, ko, vo = _expander()(tdb, q, k, v)
    return (qo, ko, vo, bias)


# speedup vs baseline: 8.5529x; 1.0114x over previous
import functools
import math

import jax
import jax.numpy as jnp
from jax import lax
from jax.experimental import pallas as pl
from jax.experimental.pallas import tpu as pltpu
from jax.experimental.pallas import tpu_sc as plsc

NUM_BUCKETS = 32
MAX_DISTANCE = 128
N_HEADS = 16

Q_LEN = 2048
K_LEN = 2048
N_RHO = 16
N_K0 = 16
SLAB_W = 128 * (N_K0 - 1) + K_LEN
C_SRC = 4224
CHUNK_R = 32
NCHUNK_PER_ARR = 4


def _table_body(wt_ref, out_ref, vtab_ref):
    c = lax.broadcasted_iota(jnp.int32, (1, C_SRC), 1)
    d = c - (Q_LEN - 1)
    nb = NUM_BUCKETS // 2
    max_exact = nb // 2
    bucket = jnp.where(d > 0, nb, 0)
    r = jnp.abs(d)
    is_small = r < max_exact
    rp_safe = jnp.maximum(r, 1).astype(jnp.float32)
    large = max_exact + (
        jnp.log(rp_safe / max_exact)
        / math.log(MAX_DISTANCE / max_exact)
        * (nb - max_exact)
    ).astype(jnp.int32)
    large = jnp.minimum(large, nb - 1)
    bucket = bucket + jnp.where(is_small, r, large)

    bkt = jnp.broadcast_to(bucket, (N_HEADS, C_SRC))
    vtab = jnp.zeros((N_HEADS, C_SRC), jnp.float32)
    for b in range(NUM_BUCKETS):
        vtab = jnp.where(bkt == b, wt_ref[:, b : b + 1], vtab)
    vtab_ref[...] = vtab
    for p in range(N_RHO):
        rho = 8 * p + 7
        for rr in range(8):
            out_ref[:, p, rr, :] = vtab_ref[:, rho - rr : rho - rr + SLAB_W]


def _build_table(W):
    wt = W.T
    return pl.pallas_call(
        _table_body,
        out_shape=jax.ShapeDtypeStruct((N_HEADS, N_RHO, 8, SLAB_W), jnp.float32),
        scratch_shapes=[pltpu.VMEM((N_HEADS, C_SRC), jnp.float32)],
    )(wt)


@functools.lru_cache(maxsize=1)
def _expander():
    mesh = plsc.VectorSubcoreMesh(core_axis_name="c", subcore_axis_name="s")

    @functools.partial(
        pl.kernel,
        mesh=mesh,
        out_type=jax.ShapeDtypeStruct((1, N_HEADS, Q_LEN, K_LEN), jnp.float32),
        scratch_types=[
            pltpu.VMEM((8, SLAB_W), jnp.float32),
            pltpu.SemaphoreType.DMA,
        ],
    )
    def expand(tdb_hbm, out_hbm, slab_ref, sem):
        wid = lax.axis_index("s") * 2 + lax.axis_index("c")
        for n in range(8):
            t = wid * 8 + n
            h = t // N_RHO
            pidx = t % N_RHO
            pltpu.sync_copy(tdb_hbm.at[h, pidx], slab_ref)
            copies = []
            for k0 in range(N_K0):
                i0 = pl.multiple_of(8 * (255 - pidx - 16 * k0), 8)
                copies.append(
                    pltpu.async_copy(
                        slab_ref.at[:, pl.ds(128 * k0, K_LEN)],
                        out_hbm.at[0, h, pl.ds(i0, 8), :],
                        sem,
                    )
                )
            for cp in copies:
                cp.wait()

    return expand


def _copy_body(q_ref, k_ref, v_ref, qo_ref, ko_ref, vo_ref):
    qo_ref[...] = q_ref[...]
    ko_ref[...] = k_ref[...]
    vo_ref[...] = v_ref[...]


def _copy_qkv(q, k, v):
    n_blk = 16
    blk = Q_LEN // n_blk
    spec = pl.BlockSpec((2, blk, 1024), lambda i: (0, i, 0))
    sds = jax.ShapeDtypeStruct(q.shape, q.dtype)
    return pl.pallas_call(
        _copy_body,
        grid=(n_blk,),
        in_specs=[spec, spec, spec],
        out_specs=[spec, spec, spec],
        out_shape=[sds, sds, sds],
    )(q, k, v)


def kernel(q, k, v, W):
    tdb = _build_table(W)
    bias = _expander()(tdb)
    qo, ko, vo = _copy_qkv(q, k, v)
    return (qo, ko, vo, bias)


# speedup vs baseline: 8.7791x; 1.0265x over previous
import functools
import math

import jax
import jax.numpy as jnp
from jax import lax
from jax.experimental import pallas as pl
from jax.experimental.pallas import tpu as pltpu
from jax.experimental.pallas import tpu_sc as plsc

NUM_BUCKETS = 32
MAX_DISTANCE = 128
N_HEADS = 16

Q_LEN = 2048
K_LEN = 2048
N_RHO = 16
N_K0 = 16
SLAB_W = 128 * (N_K0 - 1) + K_LEN
C_SRC = 4224
CHUNK_R = 32
NCHUNK_PER_ARR = 4


def _table_body(wt_ref, out_ref, vtab_ref):
    c = lax.broadcasted_iota(jnp.int32, (1, C_SRC), 1)
    d = c - (Q_LEN - 1)
    nb = NUM_BUCKETS // 2
    max_exact = nb // 2
    bucket = jnp.where(d > 0, nb, 0)
    r = jnp.abs(d)
    is_small = r < max_exact
    rp_safe = jnp.maximum(r, 1).astype(jnp.float32)
    large = max_exact + (
        jnp.log(rp_safe / max_exact)
        / math.log(MAX_DISTANCE / max_exact)
        * (nb - max_exact)
    ).astype(jnp.int32)
    large = jnp.minimum(large, nb - 1)
    bucket = bucket + jnp.where(is_small, r, large)

    bkt = jnp.broadcast_to(bucket, (N_HEADS, C_SRC))
    vtab = jnp.zeros((N_HEADS, C_SRC), jnp.float32)
    for b in range(NUM_BUCKETS):
        vtab = jnp.where(bkt == b, wt_ref[:, b : b + 1], vtab)
    vtab_ref[...] = vtab
    for p in range(N_RHO):
        rho = 8 * p + 7
        for rr in range(8):
            out_ref[:, p, rr, :] = vtab_ref[:, rho - rr : rho - rr + SLAB_W]


def _build_table(W):
    wt = W.T
    return pl.pallas_call(
        _table_body,
        out_shape=jax.ShapeDtypeStruct((N_HEADS, N_RHO, 8, SLAB_W), jnp.float32),
        scratch_shapes=[pltpu.VMEM((N_HEADS, C_SRC), jnp.float32)],
    )(wt)


@functools.lru_cache(maxsize=1)
def _expander():
    mesh = plsc.VectorSubcoreMesh(core_axis_name="c", subcore_axis_name="s")

    @functools.partial(
        pl.kernel,
        mesh=mesh,
        out_type=jax.ShapeDtypeStruct((1, N_HEADS, Q_LEN, K_LEN), jnp.float32),
        scratch_types=[
            pltpu.VMEM((8, SLAB_W), jnp.float32),
            pltpu.VMEM((8, SLAB_W), jnp.float32),
            pltpu.SemaphoreType.DMA,
            pltpu.SemaphoreType.DMA,
            pltpu.SemaphoreType.DMA,
        ],
    )
    def expand(tdb_hbm, out_hbm, slab0, slab1, in_sem, sem0, sem1):
        wid = lax.axis_index("s") * 2 + lax.axis_index("c")
        slabs = (slab0, slab1)
        out_sems = (sem0, sem1)

        def task_hp(n):
            t = wid * 8 + n
            return t // N_RHO, t % N_RHO

        def stage(n):
            h, pidx = task_hp(n)
            return pltpu.async_copy(tdb_hbm.at[h, pidx], slabs[n % 2], in_sem)

        def fire_outs(n):
            h, pidx = task_hp(n)
            copies = []
            for k0 in range(N_K0):
                i0 = pl.multiple_of(8 * (255 - pidx - 16 * k0), 8)
                copies.append(
                    pltpu.async_copy(
                        slabs[n % 2].at[:, pl.ds(128 * k0, K_LEN)],
                        out_hbm.at[0, h, pl.ds(i0, 8), :],
                        out_sems[n % 2],
                    )
                )
            return copies

        ins = {0: stage(0), 1: stage(1)}
        outs = {}
        for n in range(8):
            ins[n].wait()
            outs[n] = fire_outs(n)
            if n + 2 < 8:
                for cp in outs[n]:
                    cp.wait()
                ins[n + 2] = stage(n + 2)
        for n in (6, 7):
            for cp in outs[n]:
                cp.wait()

    return expand


def _copy_body(q_ref, k_ref, v_ref, qo_ref, ko_ref, vo_ref):
    qo_ref[...] = q_ref[...]
    ko_ref[...] = k_ref[...]
    vo_ref[...] = v_ref[...]


def _copy_qkv(q, k, v):
    n_blk = 16
    blk = Q_LEN // n_blk
    spec = pl.BlockSpec((2, blk, 1024), lambda i: (0, i, 0))
    sds = jax.ShapeDtypeStruct(q.shape, q.dtype)
    return pl.pallas_call(
        _copy_body,
        grid=(n_blk,),
        in_specs=[spec, spec, spec],
        out_specs=[spec, spec, spec],
        out_shape=[sds, sds, sds],
    )(q, k, v)


def kernel(q, k, v, W):
    tdb = _build_table(W)
    bias = _expander()(tdb)
    qo, ko, vo = _copy_qkv(q, k, v)
    return (qo, ko, vo, bias)


# speedup vs baseline: 8.8125x; 1.0038x over previous
import functools
import math

import jax
import jax.numpy as jnp
from jax import lax
from jax.experimental import pallas as pl
from jax.experimental.pallas import tpu as pltpu
from jax.experimental.pallas import tpu_sc as plsc

NUM_BUCKETS = 32
MAX_DISTANCE = 128
N_HEADS = 16

Q_LEN = 2048
K_LEN = 2048
N_RHO = 16
N_K0 = 16
SLAB_W = 128 * (N_K0 - 1) + K_LEN
C_SRC = 4224
CHUNK_R = 32
NCHUNK_PER_ARR = 4


def _table_body(wt_ref, out_ref, vtab_ref):
    c = lax.broadcasted_iota(jnp.int32, (1, C_SRC), 1)
    d = c - (Q_LEN - 1)
    nb = NUM_BUCKETS // 2
    max_exact = nb // 2
    bucket = jnp.where(d > 0, nb, 0)
    r = jnp.abs(d)
    is_small = r < max_exact
    rp_safe = jnp.maximum(r, 1).astype(jnp.float32)
    large = max_exact + (
        jnp.log(rp_safe / max_exact)
        / math.log(MAX_DISTANCE / max_exact)
        * (nb - max_exact)
    ).astype(jnp.int32)
    large = jnp.minimum(large, nb - 1)
    bucket = bucket + jnp.where(is_small, r, large)

    bkt = jnp.broadcast_to(bucket, (N_HEADS, C_SRC))
    vtab = jnp.zeros((N_HEADS, C_SRC), jnp.float32)
    for b in range(NUM_BUCKETS):
        vtab = jnp.where(bkt == b, wt_ref[:, b : b + 1], vtab)
    vtab_ref[...] = vtab
    for p in range(N_RHO):
        rho = 8 * p + 7
        for rr in range(8):
            out_ref[:, p, rr, :] = vtab_ref[:, rho - rr : rho - rr + SLAB_W]


def _build_table(W):
    wt = W.T
    return pl.pallas_call(
        _table_body,
        out_shape=jax.ShapeDtypeStruct((N_HEADS, N_RHO, 8, SLAB_W), jnp.float32),
        scratch_shapes=[pltpu.VMEM((N_HEADS, C_SRC), jnp.float32)],
    )(wt)


@functools.lru_cache(maxsize=1)
def _expander():
    mesh = plsc.VectorSubcoreMesh(core_axis_name="c", subcore_axis_name="s")

    @functools.partial(
        pl.kernel,
        mesh=mesh,
        out_type=jax.ShapeDtypeStruct((1, N_HEADS, Q_LEN, K_LEN), jnp.float32),
        scratch_types=[
            pltpu.VMEM((8, SLAB_W), jnp.float32),
            pltpu.VMEM((8, SLAB_W), jnp.float32),
            pltpu.SemaphoreType.DMA,
            pltpu.SemaphoreType.DMA,
            pltpu.SemaphoreType.DMA,
        ],
    )
    def expand(tdb_hbm, out_hbm, slab0, slab1, in_sem, sem0, sem1):
        wid = lax.axis_index("s") * 2 + lax.axis_index("c")
        slabs = (slab0, slab1)
        out_sems = (sem0, sem1)

        def task_hp(n):
            t = wid * 8 + n
            return t // N_RHO, t % N_RHO

        def stage(n):
            h, pidx = task_hp(n)
            return pltpu.async_copy(tdb_hbm.at[h, pidx], slabs[n % 2], in_sem)

        def fire_outs(n):
            h, pidx = task_hp(n)
            copies = []
            for k0 in range(N_K0):
                i0 = pl.multiple_of(8 * (255 - pidx - 16 * k0), 8)
                copies.append(
                    pltpu.async_copy(
                        slabs[n % 2].at[:, pl.ds(128 * k0, K_LEN)],
                        out_hbm.at[0, h, pl.ds(i0, 8), :],
                        out_sems[n % 2],
                    )
                )
            return copies

        ins = {0: stage(0), 1: stage(1)}
        outs = {}
        for n in range(8):
            ins[n].wait()
            outs[n] = fire_outs(n)
            if n + 2 < 8:
                for cp in outs[n]:
                    cp.wait()
                ins[n + 2] = stage(n + 2)
        for n in (6, 7):
            for cp in outs[n]:
                cp.wait()

    return expand


def _copy_body(q_ref, k_ref, v_ref, qo_ref, ko_ref, vo_ref):
    qo_ref[...] = q_ref[...]
    ko_ref[...] = k_ref[...]
    vo_ref[...] = v_ref[...]


def _copy_qkv(q, k, v):
    n_blk = 16
    blk = Q_LEN // n_blk
    spec = pl.BlockSpec((2, blk, 1024), lambda i: (0, i, 0))
    sds = jax.ShapeDtypeStruct(q.shape, q.dtype)
    return pl.pallas_call(
        _copy_body,
        grid=(n_blk,),
        in_specs=[spec, spec, spec],
        out_specs=[spec, spec, spec],
        out_shape=[sds, sds, sds],
    )(q, k, v)


def kernel(q, k, v, W):
    tdb = _build_table(W)
    bias = _expander()(tdb)
    return (q, k, v, bias)
